# Initial kernel scaffold; baseline (speedup 1.0000x reference)
import jax
import jax.numpy as jnp
from jax.experimental import pallas as pl


def kernel(x, edge_index, hop_pos, hop_neg, W1, b1, W_mu, W_sigma):
    raise NotImplementedError("write your pallas kernel here")


# speedup vs baseline: 2.4516x; 2.4516x over previous
import functools

import jax
import jax.numpy as jnp
from jax import lax
from jax.experimental import pallas as pl
from jax.experimental.pallas import tpu as pltpu
from jax.experimental.pallas import tpu_sc as plsc

NC = 2
NS = 16
W = 144
CK = 80


def _seg_body(xaug, ei, zrows, out, acc, sidx, didx, rows, sem):
    c = lax.axis_index("c")
    s = lax.axis_index("s")
    n = acc.shape[0]
    npt = n // NS
    pltpu.sync_copy(zrows, acc.at[pl.ds(s * npt, npt)])
    plsc.subcore_barrier()
    e = ei.shape[1]
    ept = e // (NC * NS)
    base = (c * NS + s) * ept

    def body(j, carry):
        b = base + j * CK
        pltpu.sync_copy(ei.at[0, pl.ds(b, CK)], sidx)
        pltpu.sync_copy(ei.at[1, pl.ds(b, CK)], didx)
        pltpu.async_copy(xaug.at[sidx], rows, sem).wait()
        pltpu.sync_copy(rows, acc.at[didx], add=True)
        return carry

    lax.fori_loop(0, ept // CK, body, 0)
    plsc.subcore_barrier()
    pltpu.sync_copy(acc.at[pl.ds(s * npt, npt)],
                    out.at[c, pl.ds(s * npt, npt)])


def _enc_body(xr, pr, w1r, b1r, wmur, wsgr, ar, br):
    msg = pr[0] + pr[1]
    deg = jnp.maximum(msg[:, 128:129], 1.0)
    agg = msg[:, :128] / deg
    xa = xr[...] + agg
    h = jnp.maximum(
        jnp.dot(xa, w1r[...], preferred_element_type=jnp.float32) + b1r[...],
        0.0)
    mu = jnp.dot(h, wmur[...], preferred_element_type=jnp.float32)
    z = jnp.dot(h, wsgr[...], preferred_element_type=jnp.float32)
    sig = (jnp.where(z > 0, z, jnp.exp(z) - 1.0) + 1.0) + 1e-14
    pinv = 1.0 / sig
    q = mu * pinv
    slog = jnp.sum(jnp.log(sig), axis=1, keepdims=True)
    rmu = jnp.sum(mu * q, axis=1, keepdims=True)
    zs = jnp.zeros((xa.shape[0], W - 129), jnp.float32)
    ar[...] = jnp.concatenate([pinv, q, rmu + slog, zs], axis=1)
    br[...] = jnp.concatenate([sig + mu * mu, -2.0 * mu, slog, zs], axis=1)


def _pair_body(atab, btab, hp, hn, out, aidx, bidx, arows, brows, accv, sem):
    c = lax.axis_index("c")
    s = lax.axis_index("s")
    e = hp.shape[1]
    ept = e // (NC * NS)
    base = (c * NS + s) * ept
    rows16 = [jnp.arange(16, dtype=jnp.int32) + g * 16 for g in range(CK // 16)]

    def make_body(hop, is_pos):
        def body(j, acc):
            b = base + j * CK
            pltpu.sync_copy(hop.at[0, pl.ds(b, CK)], aidx)
            pltpu.sync_copy(hop.at[1, pl.ds(b, CK)], bidx)
            cpa = pltpu.async_copy(atab.at[aidx], arows, sem)
            cpb = pltpu.async_copy(btab.at[bidx], brows, sem)
            cpa.wait()
            cpb.wait()
            for g in range(CK // 16):
                r = rows16[g]
                dacc = jnp.zeros((16,), jnp.float32)
                for l in range(128):
                    col = jnp.full((16,), l, jnp.int32)
                    va = plsc.load_gather(arows, [r, col])
                    vb = plsc.load_gather(brows, [r, col])
                    dacc = dacc + va * vb
                col = jnp.full((16,), 128, jnp.int32)
                rsx = plsc.load_gather(arows, [r, col])
                sy = plsc.load_gather(brows, [r, col])
                en = 0.5 * (dacc + rsx - sy) - 32.0
                if is_pos:
                    acc = acc + en * en
                else:
                    acc = acc + jnp.exp(-en)
            return acc

        return body

    acc = jnp.zeros((16,), jnp.float32)
    acc = lax.fori_loop(0, ept // CK, make_body(hp, True), acc)
    acc = lax.fori_loop(0, ept // CK, make_body(hn, False), acc)
    accv[...] = acc
    pltpu.sync_copy(accv, out.at[c, s])


@functools.lru_cache(maxsize=None)
def _build(n, e, d, lout):
    mesh = plsc.VectorSubcoreMesh(core_axis_name="c", subcore_axis_name="s",
                                  num_cores=NC, num_subcores=NS)
    sc_params = pltpu.CompilerParams(use_tc_tiling_on_sc=False,
                                     needs_layout_passes=False)
    seg = pl.kernel(
        _seg_body,
        out_type=jax.ShapeDtypeStruct((NC, n, W), jnp.float32),
        mesh=mesh,
        compiler_params=sc_params,
        scratch_types=[
            pltpu.VMEM_SHARED((n, W), jnp.float32),
            pltpu.VMEM((CK,), jnp.int32),
            pltpu.VMEM((CK,), jnp.int32),
            pltpu.VMEM((CK, W), jnp.float32),
            pltpu.SemaphoreType.DMA,
        ],
    )

    rb = 1000
    grid = n // rb
    enc = pl.pallas_call(
        _enc_body,
        grid=(grid,),
        in_specs=[
            pl.BlockSpec((rb, d), lambda i: (i, 0)),
            pl.BlockSpec((NC, rb, W), lambda i: (0, i, 0)),
            pl.BlockSpec((d, d), lambda i: (0, 0)),
            pl.BlockSpec((1, d), lambda i: (0, 0)),
            pl.BlockSpec((d, lout), lambda i: (0, 0)),
            pl.BlockSpec((d, lout), lambda i: (0, 0)),
        ],
        out_specs=[
            pl.BlockSpec((rb, W), lambda i: (i, 0)),
            pl.BlockSpec((rb, W), lambda i: (i, 0)),
        ],
        out_shape=[
            jax.ShapeDtypeStruct((n, W), jnp.float32),
            jax.ShapeDtypeStruct((n, W), jnp.float32),
        ],
    )

    pair = pl.kernel(
        _pair_body,
        out_type=jax.ShapeDtypeStruct((NC, NS, 16), jnp.float32),
        mesh=mesh,
        compiler_params=sc_params,
        scratch_types=[
            pltpu.VMEM((CK,), jnp.int32),
            pltpu.VMEM((CK,), jnp.int32),
            pltpu.VMEM((CK, W), jnp.float32),
            pltpu.VMEM((CK, W), jnp.float32),
            pltpu.VMEM((16,), jnp.float32),
            pltpu.SemaphoreType.DMA,
        ],
    )
    return seg, enc, pair


def kernel(x, edge_index, hop_pos, hop_neg, W1, b1, W_mu, W_sigma):
    n, d = x.shape
    e = edge_index.shape[1]
    lout = W_mu.shape[1]
    seg, enc, pair = _build(n, e, d, lout)
    xaug = jnp.concatenate(
        [x, jnp.ones((n, 1), jnp.float32), jnp.zeros((n, W - d - 1), jnp.float32)],
        axis=1)
    zrows = jnp.zeros((n // NS, W), jnp.float32)
    partials = seg(xaug, edge_index, zrows)
    a_tab, b_tab = enc(x, partials, W1, b1.reshape(1, -1), W_mu, W_sigma)
    psums = pair(a_tab, b_tab, hop_pos, hop_neg)
    return jnp.sum(psums) / e
